# Initial kernel scaffold; baseline (speedup 1.0000x reference)
import jax
import jax.numpy as jnp
from jax.experimental import pallas as pl


def kernel(x_feat, edge_attr, bases, edge_index, W1, b1, W2, b2, g1, be1, W3, b3, g2, be2):
    raise NotImplementedError("write your pallas kernel here")


# speedup vs baseline: 3.8008x; 3.8008x over previous
import functools

import jax
import jax.numpy as jnp
from jax import lax
from jax.experimental import pallas as pl
from jax.experimental.pallas import tpu as pltpu
from jax.experimental.pallas import tpu_sc as plsc

N_NODES = 10000
N_EDGES = 320000
D = 128

NC = 2
NS = 16
NW = NC * NS

CHUNK = 128
N_CHUNKS = N_EDGES // CHUNK
BASE_CH = N_CHUNKS // NW
EXTRA = N_CHUNKS % NW
MAXCH = 80
NPAD = 10240
STRIPE = NPAD // NS

_mesh = lambda: plsc.VectorSubcoreMesh(core_axis_name="c", subcore_axis_name="s")


def _worker_range(wid):
    nch = BASE_CH + jnp.where(wid < EXTRA, 1, 0)
    start = wid * BASE_CH + jnp.minimum(wid, EXTRA)
    return start, nch


def _xw_body(x_ref, w_ref, o_ref):
    o_ref[...] = jnp.dot(x_ref[...], w_ref[...], preferred_element_type=jnp.float32)


def _project_nodes(x_feat, w1t):
    return pl.pallas_call(
        _xw_body,
        out_shape=jax.ShapeDtypeStruct((N_NODES, D), jnp.float32),
    )(x_feat, w1t)


EB = 4000


def _gelu(x):
    return 0.5 * x * (1.0 + lax.erf(x * 0.7071067811865476))


def _combine_body(e_ref, g_ref, bs_ref, w_ref, b1_ref, o_ref):
    pre = jnp.dot(e_ref[...], w_ref[...], preferred_element_type=jnp.float32)
    pre = pre + g_ref[...] + b1_ref[...]
    o_ref[...] = _gelu(pre) * bs_ref[...]


def _combine(edge_attr, g, bases, w1t, b1_row):
    blk = lambda i: (i, 0)
    fix = lambda i: (0, 0)
    return pl.pallas_call(
        _combine_body,
        grid=(N_EDGES // EB,),
        in_specs=[
            pl.BlockSpec((EB, D), blk),
            pl.BlockSpec((EB, D), blk),
            pl.BlockSpec((EB, D), blk),
            pl.BlockSpec((D, D), fix),
            pl.BlockSpec((1, D), fix),
        ],
        out_specs=pl.BlockSpec((EB, D), blk),
        out_shape=jax.ShapeDtypeStruct((N_EDGES, D), jnp.float32),
    )(edge_attr, g, bases, w1t, b1_row)


def _ffn_body(x_ref, a_ref, w2_ref, b2_ref, g1_ref, be1_ref,
              w3_ref, b3_ref, g2_ref, be2_ref, o_ref):
    x = x_ref[...] + a_ref[0, :N_NODES] + a_ref[1, :N_NODES]
    h = jnp.dot(x, w2_ref[...], preferred_element_type=jnp.float32) + b2_ref[...]
    mean = jnp.mean(h, axis=0, keepdims=True)
    var = jnp.mean((h - mean) ** 2, axis=0, keepdims=True)
    h = (h - mean) / jnp.sqrt(var + 1e-5) * g1_ref[...] + be1_ref[...]
    h = _gelu(h)
    h = jnp.dot(h, w3_ref[...], preferred_element_type=jnp.float32) + b3_ref[...]
    mean = jnp.mean(h, axis=0, keepdims=True)
    var = jnp.mean((h - mean) ** 2, axis=0, keepdims=True)
    h = (h - mean) / jnp.sqrt(var + 1e-5) * g2_ref[...] + be2_ref[...]
    h = _gelu(h)
    o_ref[...] = x + h


def _ffn(x_feat, acc, w2t, b2r, g1r, be1r, w3t, b3r, g2r, be2r):
    return pl.pallas_call(
        _ffn_body,
        out_shape=jax.ShapeDtypeStruct((N_NODES, D), jnp.float32),
    )(x_feat, acc, w2t, b2r, g1r, be1r, w3t, b3r, g2r, be2r)


def _gather_kernel(table_hbm, idx_hbm, out_hbm, idx_v, rows_v, sem):
    cid = lax.axis_index("c")
    sid = lax.axis_index("s")
    wid = sid * NC + cid
    start, nch = _worker_range(wid)
    pltpu.sync_copy(idx_hbm.at[wid], idx_v)

    def body(j, carry):
        ofs = pl.multiple_of((start + j) * CHUNK, CHUNK)
        pltpu.async_copy(table_hbm.at[idx_v.at[j]], rows_v, sem).wait()
        pltpu.sync_copy(rows_v, out_hbm.at[pl.ds(ofs, CHUNK)])
        return carry

    lax.fori_loop(0, nch, body, 0)


def _gather(table, idx_pad):
    k = functools.partial(
        pl.kernel,
        out_type=jax.ShapeDtypeStruct((N_EDGES, D), jnp.float32),
        mesh=_mesh(),
        scratch_types=[
            pltpu.VMEM((MAXCH, CHUNK), jnp.int32),
            pltpu.VMEM((CHUNK, D), jnp.float32),
            pltpu.SemaphoreType.DMA,
        ],
    )(_gather_kernel)
    return k(table, idx_pad)


def _scatter_kernel(v_hbm, dst_hbm, zeros_hbm, out_hbm, idx_v, rows_v, acc_sh, sem):
    cid = lax.axis_index("c")
    sid = lax.axis_index("s")
    wid = sid * NC + cid
    pltpu.sync_copy(zeros_hbm.at[pl.ds(sid * STRIPE, STRIPE)],
                    acc_sh.at[pl.ds(sid * STRIPE, STRIPE)])
    plsc.subcore_barrier()

    start, nch = _worker_range(wid)
    pltpu.sync_copy(dst_hbm.at[wid], idx_v)

    def body(j, carry):
        ofs = pl.multiple_of((start + j) * CHUNK, CHUNK)
        pltpu.sync_copy(v_hbm.at[pl.ds(ofs, CHUNK)], rows_v)
        pltpu.sync_copy(rows_v, acc_sh.at[idx_v.at[j]], add=True)
        return carry

    lax.fori_loop(0, nch, body, 0)
    plsc.subcore_barrier()
    pltpu.sync_copy(acc_sh.at[pl.ds(sid * STRIPE, STRIPE)],
                    out_hbm.at[cid, pl.ds(sid * STRIPE, STRIPE)])


def _scatter(v, dst_pad, zeros):
    k = functools.partial(
        pl.kernel,
        out_type=jax.ShapeDtypeStruct((NC, NPAD, D), jnp.float32),
        mesh=_mesh(),
        scratch_types=[
            pltpu.VMEM((MAXCH, CHUNK), jnp.int32),
            pltpu.VMEM((CHUNK, D), jnp.float32),
            pltpu.VMEM_SHARED((NPAD, D), jnp.float32),
            pltpu.SemaphoreType.DMA,
        ],
    )(_scatter_kernel)
    return k(v, dst_pad, zeros)


import numpy as _np

_W_STARTS = _np.array([w * BASE_CH + min(w, EXTRA) for w in range(NW)])
_W_ROWS = _np.minimum(_W_STARTS[:, None] + _np.arange(MAXCH)[None, :], N_CHUNKS - 1)


def _pad_idx(idx):
    idx2 = idx.reshape(N_CHUNKS, CHUNK)
    return jnp.take(idx2, jnp.asarray(_W_ROWS), axis=0)


def kernel(x_feat, edge_attr, bases, edge_index, W1, b1, W2, b2, g1, be1, W3, b3, g2, be2):
    src_pad = _pad_idx(edge_index[0])
    dst_pad = _pad_idx(edge_index[1])
    zeros = jnp.zeros((NPAD, D), jnp.float32)

    xw = _project_nodes(x_feat, W1.T)
    g = _gather(xw, src_pad)
    v = _combine(edge_attr, g, bases, W1.T, b1.reshape(1, D))
    acc = _scatter(v, dst_pad, zeros)
    out = _ffn(x_feat, acc, W2.T, b2.reshape(1, D), g1.reshape(1, D),
               be1.reshape(1, D), W3.T, b3.reshape(1, D), g2.reshape(1, D),
               be2.reshape(1, D))
    return out


# speedup vs baseline: 4.7151x; 1.2405x over previous
import functools

import jax
import jax.numpy as jnp
from jax import lax
from jax.experimental import pallas as pl
from jax.experimental.pallas import tpu as pltpu
from jax.experimental.pallas import tpu_sc as plsc

N_NODES = 10000
N_EDGES = 320000
D = 128

NC = 2
NS = 16
NW = NC * NS

CHUNK = 128
N_CHUNKS = N_EDGES // CHUNK
BASE_CH = N_CHUNKS // NW
EXTRA = N_CHUNKS % NW
MAXCH = 80
NPAD = 10240
STRIPE = NPAD // NS

_mesh = lambda: plsc.VectorSubcoreMesh(core_axis_name="c", subcore_axis_name="s")


def _worker_range(wid):
    nch = BASE_CH + jnp.where(wid < EXTRA, 1, 0)
    start = wid * BASE_CH + jnp.minimum(wid, EXTRA)
    return start, nch


def _xw_body(x_ref, w_ref, o_ref):
    o_ref[...] = jnp.dot(x_ref[...], w_ref[...], preferred_element_type=jnp.float32)


def _project_nodes(x_feat, w1t):
    return pl.pallas_call(
        _xw_body,
        out_shape=jax.ShapeDtypeStruct((N_NODES, D), jnp.float32),
    )(x_feat, w1t)


EB = 4000


def _gelu(x):
    return 0.5 * x * (1.0 + lax.erf(x * 0.7071067811865476))


def _combine_body(e_ref, g_ref, bs_ref, w_ref, b1_ref, o_ref):
    pre = jnp.dot(e_ref[...], w_ref[...], preferred_element_type=jnp.float32)
    pre = pre + g_ref[...] + b1_ref[...]
    o_ref[...] = _gelu(pre) * bs_ref[...]


def _combine(edge_attr, g, bases, w1t, b1_row):
    blk = lambda i: (i, 0)
    fix = lambda i: (0, 0)
    return pl.pallas_call(
        _combine_body,
        grid=(N_EDGES // EB,),
        in_specs=[
            pl.BlockSpec((EB, D), blk),
            pl.BlockSpec((EB, D), blk),
            pl.BlockSpec((EB, D), blk),
            pl.BlockSpec((D, D), fix),
            pl.BlockSpec((1, D), fix),
        ],
        out_specs=pl.BlockSpec((EB, D), blk),
        out_shape=jax.ShapeDtypeStruct((N_EDGES, D), jnp.float32),
    )(edge_attr, g, bases, w1t, b1_row)


def _ffn_body(x_ref, a_ref, w2_ref, b2_ref, g1_ref, be1_ref,
              w3_ref, b3_ref, g2_ref, be2_ref, o_ref):
    x = x_ref[...] + a_ref[0, :N_NODES] + a_ref[1, :N_NODES]
    h = jnp.dot(x, w2_ref[...], preferred_element_type=jnp.float32) + b2_ref[...]
    mean = jnp.mean(h, axis=0, keepdims=True)
    var = jnp.mean((h - mean) ** 2, axis=0, keepdims=True)
    h = (h - mean) / jnp.sqrt(var + 1e-5) * g1_ref[...] + be1_ref[...]
    h = _gelu(h)
    h = jnp.dot(h, w3_ref[...], preferred_element_type=jnp.float32) + b3_ref[...]
    mean = jnp.mean(h, axis=0, keepdims=True)
    var = jnp.mean((h - mean) ** 2, axis=0, keepdims=True)
    h = (h - mean) / jnp.sqrt(var + 1e-5) * g2_ref[...] + be2_ref[...]
    h = _gelu(h)
    o_ref[...] = x + h


def _ffn(x_feat, acc, w2t, b2r, g1r, be1r, w3t, b3r, g2r, be2r):
    return pl.pallas_call(
        _ffn_body,
        out_shape=jax.ShapeDtypeStruct((N_NODES, D), jnp.float32),
    )(x_feat, acc, w2t, b2r, g1r, be1r, w3t, b3r, g2r, be2r)


KB = 3
BROWS = KB * CHUNK
NFULL = BASE_CH // KB
NPAIR = NFULL // 2
NPAIR_S = BASE_CH // 2


def _gather_kernel(table_hbm, idx_hbm, out_hbm, idx_v, rows_a, rows_b, gsa, gsb):
    cid = lax.axis_index("c")
    sid = lax.axis_index("s")
    wid = sid * NC + cid
    start, nch = _worker_range(wid)
    pltpu.sync_copy(idx_hbm.at[wid], idx_v)

    def issue(b, rows, sem):
        for k in range(KB):
            pltpu.async_copy(table_hbm.at[idx_v.at[b * KB + k]],
                             rows.at[pl.ds(k * CHUNK, CHUNK)], sem)

    def wait(rows, sem):
        pltpu.make_async_copy(out_hbm.at[pl.ds(0, BROWS)], rows, sem).wait()

    def store(b, rows):
        ofs = pl.multiple_of((start + b * KB) * CHUNK, CHUNK)
        pltpu.sync_copy(rows, out_hbm.at[pl.ds(ofs, BROWS)])

    issue(0, rows_a, gsa)

    def body(i, carry):
        wait(rows_a, gsa)
        issue(2 * i + 1, rows_b, gsb)
        store(2 * i, rows_a)

        @pl.when(i < NPAIR - 1)
        def _():
            issue(2 * i + 2, rows_a, gsa)

        wait(rows_b, gsb)
        store(2 * i + 1, rows_b)
        return carry

    lax.fori_loop(0, NPAIR, body, 0)

    @pl.when(nch > NFULL * KB)
    def _():
        pltpu.async_copy(table_hbm.at[idx_v.at[NFULL * KB]],
                         rows_a.at[pl.ds(0, CHUNK)], gsa).wait()
        ofs = pl.multiple_of((start + NFULL * KB) * CHUNK, CHUNK)
        pltpu.sync_copy(rows_a.at[pl.ds(0, CHUNK)], out_hbm.at[pl.ds(ofs, CHUNK)])


def _gather(table, idx_pad):
    k = functools.partial(
        pl.kernel,
        out_type=jax.ShapeDtypeStruct((N_EDGES, D), jnp.float32),
        mesh=_mesh(),
        scratch_types=[
            pltpu.VMEM((MAXCH, CHUNK), jnp.int32),
            pltpu.VMEM((BROWS, D), jnp.float32),
            pltpu.VMEM((BROWS, D), jnp.float32),
            pltpu.SemaphoreType.DMA,
            pltpu.SemaphoreType.DMA,
        ],
    )(_gather_kernel)
    return k(table, idx_pad)


def _scatter_kernel(v_hbm, dst_hbm, zeros_hbm, out_hbm, idx_v, rows_a, rows_b, acc_sh, lsa, lsb):
    cid = lax.axis_index("c")
    sid = lax.axis_index("s")
    wid = sid * NC + cid
    pltpu.sync_copy(zeros_hbm.at[pl.ds(sid * STRIPE, STRIPE)],
                    acc_sh.at[pl.ds(sid * STRIPE, STRIPE)])
    plsc.subcore_barrier()

    start, nch = _worker_range(wid)
    pltpu.sync_copy(dst_hbm.at[wid], idx_v)

    def issue(j, rows, sem):
        ofs = pl.multiple_of((start + j) * CHUNK, CHUNK)
        pltpu.async_copy(v_hbm.at[pl.ds(ofs, CHUNK)], rows, sem)

    def wait(rows, sem):
        pltpu.make_async_copy(v_hbm.at[pl.ds(0, CHUNK)], rows, sem).wait()

    def scat(j, rows):
        pltpu.sync_copy(rows, acc_sh.at[idx_v.at[j]], add=True)

    issue(0, rows_a, lsa)

    def body(i, carry):
        wait(rows_a, lsa)
        issue(2 * i + 1, rows_b, lsb)
        scat(2 * i, rows_a)

        @pl.when(i < NPAIR_S - 1)
        def _():
            issue(2 * i + 2, rows_a, lsa)

        wait(rows_b, lsb)
        scat(2 * i + 1, rows_b)
        return carry

    lax.fori_loop(0, NPAIR_S, body, 0)

    @pl.when(nch > BASE_CH)
    def _():
        ofs = pl.multiple_of((start + BASE_CH) * CHUNK, CHUNK)
        pltpu.sync_copy(v_hbm.at[pl.ds(ofs, CHUNK)], rows_a)
        pltpu.sync_copy(rows_a, acc_sh.at[idx_v.at[BASE_CH]], add=True)

    plsc.subcore_barrier()
    pltpu.sync_copy(acc_sh.at[pl.ds(sid * STRIPE, STRIPE)],
                    out_hbm.at[cid, pl.ds(sid * STRIPE, STRIPE)])


def _scatter(v, dst_pad, zeros):
    k = functools.partial(
        pl.kernel,
        out_type=jax.ShapeDtypeStruct((NC, NPAD, D), jnp.float32),
        mesh=_mesh(),
        scratch_types=[
            pltpu.VMEM((MAXCH, CHUNK), jnp.int32),
            pltpu.VMEM((CHUNK, D), jnp.float32),
            pltpu.VMEM((CHUNK, D), jnp.float32),
            pltpu.VMEM_SHARED((NPAD, D), jnp.float32),
            pltpu.SemaphoreType.DMA,
            pltpu.SemaphoreType.DMA,
        ],
    )(_scatter_kernel)
    return k(v, dst_pad, zeros)


import numpy as _np

_W_STARTS = _np.array([w * BASE_CH + min(w, EXTRA) for w in range(NW)])
_W_ROWS = _np.minimum(_W_STARTS[:, None] + _np.arange(MAXCH)[None, :], N_CHUNKS - 1)


def _pad_idx(idx):
    idx2 = idx.reshape(N_CHUNKS, CHUNK)
    return jnp.take(idx2, jnp.asarray(_W_ROWS), axis=0)


def kernel(x_feat, edge_attr, bases, edge_index, W1, b1, W2, b2, g1, be1, W3, b3, g2, be2):
    src_pad = _pad_idx(edge_index[0])
    dst_pad = _pad_idx(edge_index[1])
    zeros = jnp.zeros((NPAD, D), jnp.float32)

    xw = _project_nodes(x_feat, W1.T)
    g = _gather(xw, src_pad)
    v = _combine(edge_attr, g, bases, W1.T, b1.reshape(1, D))
    acc = _scatter(v, dst_pad, zeros)
    out = _ffn(x_feat, acc, W2.T, b2.reshape(1, D), g1.reshape(1, D),
               be1.reshape(1, D), W3.T, b3.reshape(1, D), g2.reshape(1, D),
               be2.reshape(1, D))
    return out
